# baseline (speedup 1.0000x reference)
import functools

import jax
import jax.numpy as jnp
from jax import lax
from jax.experimental import pallas as pl
from jax.experimental.pallas import tpu as pltpu
from jax.experimental.pallas import tpu_sc as plsc

LANES = 16
CHUNK = 80
NWORKERS = 32
SCPAD = 3 * CHUNK + LANES


def _bcast(s):
    return lax.broadcast(s, (LANES,))


def _rsqrt(v):
    i = lax.bitcast_convert_type(v, jnp.int32)
    i = jnp.int32(0x5F3759DF) - lax.shift_right_logical(i, 1)
    y = lax.bitcast_convert_type(i, jnp.float32)
    for _ in range(3):
        y = y * (1.5 - 0.5 * v * y * y)
    return y


def _make_kernel(L, N, D):
    assert D == 128 and L == 4
    G = D // LANES
    nchunks = N // CHUNK
    assert N % CHUNK == 0

    mesh = plsc.VectorSubcoreMesh(core_axis_name="c", subcore_axis_name="s")

    @functools.partial(
        pl.kernel,
        mesh=mesh,
        compiler_params=pltpu.CompilerParams(needs_layout_passes=False),
        out_type=[
            jax.ShapeDtypeStruct((N, D), jnp.float32),
            jax.ShapeDtypeStruct(((L - 1) * N,), jnp.float32),
        ],
        scratch_types=[
            pltpu.VMEM((2, L, CHUNK, D), jnp.float32),
            pltpu.VMEM((2, CHUNK, D), jnp.float32),
            pltpu.VMEM((2 * SCPAD,), jnp.float32),
            pltpu.VMEM((272,), jnp.float32),
            pltpu.VMEM((8 * CHUNK + LANES,), jnp.float32),
            pltpu.SemaphoreType.DMA((2,)),
            pltpu.SemaphoreType.DMA((2,)),
        ],
    )
    def merge_kernel(xs_hbm, wb_hbm, emb_hbm, sc_hbm,
                     inb, embb, scb, wv, stage, sem_in, sem_out):
        wid = lax.axis_index("s") * 2 + lax.axis_index("c")
        pltpu.sync_copy(wb_hbm, wv)
        w1 = [wv[pl.ds(g * LANES, LANES)] for g in range(G)]
        w2 = [wv[pl.ds(D + g * LANES, LANES)] for g in range(G)]
        bb = _bcast(wv[pl.ds(2 * D, LANES)][0])
        my_n = (nchunks + (NWORKERS - 1) - wid) // NWORKERS

        def chunk_base(i):
            return (wid + i * NWORKERS) * CHUNK

        def in_copy(i, slot):
            return pltpu.make_async_copy(
                xs_hbm.at[:, pl.ds(chunk_base(i), CHUNK)],
                inb.at[slot], sem_in.at[slot])

        def emb_copy(i, slot):
            return pltpu.make_async_copy(
                embb.at[slot], emb_hbm.at[pl.ds(chunk_base(i), CHUNK)],
                sem_out.at[slot])

        def sc_copy(i, slot, l):
            return pltpu.make_async_copy(
                scb.at[pl.ds(slot * SCPAD + l * CHUNK, CHUNK)],
                sc_hbm.at[pl.ds(l * N + chunk_base(i), CHUNK)],
                sem_out.at[slot])

        in_copy(0, 0).start()

        lane = lax.iota(jnp.int32, LANES)
        lane_last = lane == (LANES - 1)

        def tree(vs):
            while len(vs) > 1:
                nxt = [vs[i] + vs[i + 1] for i in range(0, len(vs) - 1, 2)]
                if len(vs) % 2:
                    nxt.append(vs[-1])
                vs = nxt
            return vs[0]

        def chunk_body(i, carry):
            slot = lax.rem(i, 2)

            @pl.when(i + 1 < my_n)
            def _():
                in_copy(i + 1, 1 - slot).start()

            in_copy(i, slot).wait()

            @pl.when(i >= 2)
            def _():
                emb_copy(i - 2, slot).wait()
                for l in range(L - 1):
                    sc_copy(i - 2, slot, l).wait()

            def red_body(n, c2):
                x = [[inb[slot, l, n, pl.ds(g * LANES, LANES)]
                      for g in range(G)] for l in range(L)]
                qs = [tree([x[l][g] * x[l][g] for g in range(G)])
                      for l in range(L)]
                qs.append(tree([x[0][g] * w2[g] for g in range(G)]))
                qs += [tree([x[l][g] * w1[g] for g in range(G)])
                       for l in range(1, L)]
                for q, acc in enumerate(qs):
                    plsc.store_compressed(
                        stage.at[pl.ds(q * CHUNK + n, LANES)],
                        plsc.cumsum(acc), mask=lane_last)
                return c2

            lax.fori_loop(0, CHUNK, red_body, 0)

            for g5 in range(CHUNK // LANES):
                n0 = g5 * LANES
                ssv = [stage[pl.ds(l * CHUNK + n0, LANES)] for l in range(L)]
                dqv = stage[pl.ds(L * CHUNK + n0, LANES)]
                dmv = [stage[pl.ds((L + 1 + l) * CHUNK + n0, LANES)]
                       for l in range(L - 1)]
                inv = [_rsqrt(jnp.maximum(ssv[l], 1e-24)) for l in range(L)]
                sq = dqv * inv[0]
                s = [dmv[l] * inv[l + 1] + sq + bb for l in range(L - 1)]
                s = [jnp.where(t >= 0, t, 0.01 * t) for t in s]
                mx = jnp.maximum(jnp.maximum(s[0], s[1]), s[2])
                e = [jnp.exp(t - mx) for t in s]
                den = e[0] + e[1] + e[2] + 1e-16
                a = [t / den for t in e]
                for l in range(L - 1):
                    scb[pl.ds(slot * SCPAD + l * CHUNK + n0, LANES)] = a[l]
                cv = [inv[0]] + [a[l] * inv[l + 1] for l in range(L - 1)]

                def emb_body(j, c2, n0=n0, cv=cv):
                    n = n0 + j
                    idx = lax.broadcast(j, (LANES,))
                    csp = [jnp.take_along_axis(v, idx, axis=0) for v in cv]
                    for g in range(G):
                        xg = [inb[slot, l, n, pl.ds(g * LANES, LANES)]
                              for l in range(L)]
                        embb[slot, n, pl.ds(g * LANES, LANES)] = tree(
                            [csp[l] * xg[l] for l in range(L)])
                    return c2

                lax.fori_loop(0, LANES, emb_body, 0)
            emb_copy(i, slot).start()
            for l in range(L - 1):
                sc_copy(i, slot, l).start()
            return carry

        lax.fori_loop(0, my_n, chunk_body, 0)

        @pl.when(my_n >= 2)
        def _():
            s = lax.rem(my_n, 2)
            emb_copy(my_n - 2, s).wait()
            for l in range(L - 1):
                sc_copy(my_n - 2, s, l).wait()

        s = lax.rem(my_n - 1, 2)
        emb_copy(my_n - 1, s).wait()
        for l in range(L - 1):
            sc_copy(my_n - 1, s, l).wait()

    return merge_kernel


def kernel(xs, W_att, b_att):
    L, N, D = xs.shape
    wb = jnp.concatenate(
        [W_att[:, 0], b_att, jnp.zeros((15,), jnp.float32)])
    emb, sc = _make_kernel(L, N, D)(xs, wb)
    return emb, sc


# speedup vs baseline: 1.2428x; 1.2428x over previous
import functools

import jax
import jax.numpy as jnp
from jax import lax
from jax.experimental import pallas as pl
from jax.experimental.pallas import tpu as pltpu
from jax.experimental.pallas import tpu_sc as plsc

LANES = 16
CHUNK = 80
NWORKERS = 32
SCPAD = 3 * CHUNK + LANES


def _bcast(s):
    return lax.broadcast(s, (LANES,))


def _rsqrt(v):
    i = lax.bitcast_convert_type(v, jnp.int32)
    i = jnp.int32(0x5F3759DF) - lax.shift_right_logical(i, 1)
    y = lax.bitcast_convert_type(i, jnp.float32)
    for _ in range(3):
        y = y * (1.5 - 0.5 * v * y * y)
    return y


def _make_kernel(L, N, D):
    assert D == 128 and L == 4
    G = D // LANES
    nchunks = N // CHUNK
    assert N % CHUNK == 0

    mesh = plsc.VectorSubcoreMesh(core_axis_name="c", subcore_axis_name="s")

    @functools.partial(
        pl.kernel,
        mesh=mesh,
        compiler_params=pltpu.CompilerParams(needs_layout_passes=False),
        out_type=[
            jax.ShapeDtypeStruct((N, D), jnp.float32),
            jax.ShapeDtypeStruct(((L - 1) * N,), jnp.float32),
        ],
        scratch_types=[
            pltpu.VMEM((2, L, CHUNK, D), jnp.float32),
            pltpu.VMEM((2, CHUNK, D), jnp.float32),
            pltpu.VMEM((2 * SCPAD,), jnp.float32),
            pltpu.VMEM((272,), jnp.float32),
            pltpu.VMEM((CHUNK * LANES,), jnp.float32),
            pltpu.SemaphoreType.DMA((2,)),
            pltpu.SemaphoreType.DMA((2,)),
        ],
    )
    def merge_kernel(xs_hbm, wb_hbm, emb_hbm, sc_hbm,
                     inb, embb, scb, wv, scst, sem_in, sem_out):
        wid = lax.axis_index("s") * 2 + lax.axis_index("c")
        pltpu.sync_copy(wb_hbm, wv)
        w1 = [wv[pl.ds(g * LANES, LANES)] for g in range(G)]
        w2 = [wv[pl.ds(D + g * LANES, LANES)] for g in range(G)]
        bb = _bcast(wv[pl.ds(2 * D, LANES)][0])
        my_n = (nchunks + (NWORKERS - 1) - wid) // NWORKERS

        def chunk_base(i):
            return (wid + i * NWORKERS) * CHUNK

        def in_copy(i, slot):
            return pltpu.make_async_copy(
                xs_hbm.at[:, pl.ds(chunk_base(i), CHUNK)],
                inb.at[slot], sem_in.at[slot])

        def emb_copy(i, slot):
            return pltpu.make_async_copy(
                embb.at[slot], emb_hbm.at[pl.ds(chunk_base(i), CHUNK)],
                sem_out.at[slot])

        def sc_copy(i, slot, l):
            return pltpu.make_async_copy(
                scb.at[pl.ds(slot * SCPAD + l * CHUNK, CHUNK)],
                sc_hbm.at[pl.ds(l * N + chunk_base(i), CHUNK)],
                sem_out.at[slot])

        in_copy(0, 0).start()

        lane = lax.iota(jnp.int32, LANES)
        lane_first = lane == 0

        def tree(vs):
            while len(vs) > 1:
                nxt = [vs[i] + vs[i + 1] for i in range(0, len(vs) - 1, 2)]
                if len(vs) % 2:
                    nxt.append(vs[-1])
                vs = nxt
            return vs[0]

        def chunk_body(i, carry):
            slot = lax.rem(i, 2)

            @pl.when(i + 1 < my_n)
            def _():
                in_copy(i + 1, 1 - slot).start()

            in_copy(i, slot).wait()

            @pl.when(i >= 2)
            def _():
                emb_copy(i - 2, slot).wait()
                for l in range(L - 1):
                    sc_copy(i - 2, slot, l).wait()

            def node_body(n, c2):
                x = [[inb[slot, l, n, pl.ds(g * LANES, LANES)]
                      for g in range(G)] for l in range(L)]

                def red(vs):
                    return _bcast(jnp.sum(tree(vs)))

                ss = [red([x[l][g] * x[l][g] for g in range(G)])
                      for l in range(L)]
                dq = red([x[0][g] * w2[g] for g in range(G)])
                dm = [red([x[l][g] * w1[g] for g in range(G)])
                      for l in range(1, L)]
                inv = [_rsqrt(jnp.maximum(ss[l], 1e-24)) for l in range(L)]
                sq = dq * inv[0]
                s = [dm[l] * inv[l + 1] + sq + bb for l in range(L - 1)]
                s = [jnp.where(t >= 0, t, 0.01 * t) for t in s]
                mx = jnp.maximum(jnp.maximum(s[0], s[1]), s[2])
                e = [jnp.exp(t - mx) for t in s]
                den = e[0] + e[1] + e[2] + 1e-16
                a = [t / den for t in e]
                c = [a[l] * inv[l + 1] for l in range(L - 1)]
                for g in range(G):
                    embb[slot, n, pl.ds(g * LANES, LANES)] = (
                        x[0][g] * inv[0]
                        + c[0] * x[1][g] + c[1] * x[2][g] + c[2] * x[3][g])
                av = jnp.where(lane == 0, a[0],
                               jnp.where(lane == 1, a[1], a[2]))
                plsc.store_compressed(scst.at[pl.ds(n * LANES, LANES)],
                                      av, mask=lane < 3)
                return c2

            lax.fori_loop(0, CHUNK, node_body, 0)

            for l in range(L - 1):
                for g5 in range(CHUNK // LANES):
                    vals = plsc.load_gather(
                        scst.at[pl.ds(g5 * LANES * LANES, LANES * LANES)],
                        [lane * LANES + l])
                    scb[pl.ds(slot * SCPAD + l * CHUNK + g5 * LANES,
                              LANES)] = vals

            emb_copy(i, slot).start()
            for l in range(L - 1):
                sc_copy(i, slot, l).start()
            return carry

        lax.fori_loop(0, my_n, chunk_body, 0)

        @pl.when(my_n >= 2)
        def _():
            s = lax.rem(my_n, 2)
            emb_copy(my_n - 2, s).wait()
            for l in range(L - 1):
                sc_copy(my_n - 2, s, l).wait()

        s = lax.rem(my_n - 1, 2)
        emb_copy(my_n - 1, s).wait()
        for l in range(L - 1):
            sc_copy(my_n - 1, s, l).wait()

    return merge_kernel


def kernel(xs, W_att, b_att):
    L, N, D = xs.shape
    wb = jnp.concatenate(
        [W_att[:, 0], b_att, jnp.zeros((15,), jnp.float32)])
    emb, sc = _make_kernel(L, N, D)(xs, wb)
    return emb, sc


# speedup vs baseline: 1.3045x; 1.0496x over previous
import functools

import jax
import jax.numpy as jnp
from jax import lax
from jax.experimental import pallas as pl
from jax.experimental.pallas import tpu as pltpu
from jax.experimental.pallas import tpu_sc as plsc

LANES = 16
CHUNK = 80
NWORKERS = 32
SCPAD = 3 * CHUNK + LANES


def _rsqrt(v):
    i = lax.bitcast_convert_type(v, jnp.int32)
    i = jnp.int32(0x5F3759DF) - lax.shift_right_logical(i, 1)
    y = lax.bitcast_convert_type(i, jnp.float32)
    for _ in range(2):
        y = y * (1.5 - 0.5 * v * y * y)
    return y


def _tree(vs):
    while len(vs) > 1:
        nxt = [vs[i] + vs[i + 1] for i in range(0, len(vs) - 1, 2)]
        if len(vs) % 2:
            nxt.append(vs[-1])
        vs = nxt
    return vs[0]


def _make_kernel(L, N, D):
    assert D == 128 and L == 4
    G = D // LANES
    nchunks = N // CHUNK
    assert N % CHUNK == 0

    mesh = plsc.VectorSubcoreMesh(core_axis_name="c", subcore_axis_name="s")

    @functools.partial(
        pl.kernel,
        mesh=mesh,
        compiler_params=pltpu.CompilerParams(needs_layout_passes=False),
        out_type=[
            jax.ShapeDtypeStruct((N, D), jnp.float32),
            jax.ShapeDtypeStruct(((L - 1) * N,), jnp.float32),
        ],
        scratch_types=[
            pltpu.VMEM((2, L, CHUNK, D), jnp.float32),
            pltpu.VMEM((2, CHUNK, D), jnp.float32),
            pltpu.VMEM((2 * SCPAD,), jnp.float32),
            pltpu.VMEM((272,), jnp.float32),
            pltpu.SemaphoreType.DMA((2,)),
            pltpu.SemaphoreType.DMA((2,)),
        ],
    )
    def merge_kernel(xs_hbm, wb_hbm, emb_hbm, sc_hbm,
                     inb, embb, scb, wv, sem_in, sem_out):
        wid = lax.axis_index("s") * 2 + lax.axis_index("c")
        pltpu.sync_copy(wb_hbm, wv)
        w1 = [wv[pl.ds(g * LANES, LANES)] for g in range(G)]
        w2 = [wv[pl.ds(D + g * LANES, LANES)] for g in range(G)]
        lane = lax.iota(jnp.int32, LANES)
        lane_first = lane == 0
        bb = lax.broadcast(wv[pl.ds(2 * D, LANES)][0], (LANES,))
        my_n = (nchunks + (NWORKERS - 1) - wid) // NWORKERS

        def _bcast(s):
            return lax.broadcast(s, (LANES,))

        def chunk_base(i):
            return (wid + i * NWORKERS) * CHUNK

        def in_copy(i, slot):
            return pltpu.make_async_copy(
                xs_hbm.at[:, pl.ds(chunk_base(i), CHUNK)],
                inb.at[slot], sem_in.at[slot])

        def emb_copy(i, slot):
            return pltpu.make_async_copy(
                embb.at[slot], emb_hbm.at[pl.ds(chunk_base(i), CHUNK)],
                sem_out.at[slot])

        def sc_copy(i, slot, l):
            return pltpu.make_async_copy(
                scb.at[pl.ds(slot * SCPAD + l * CHUNK, CHUNK)],
                sc_hbm.at[pl.ds(l * N + chunk_base(i), CHUNK)],
                sem_out.at[slot])

        in_copy(0, 0).start()

        def chunk_body(i, carry):
            slot = lax.rem(i, 2)

            @pl.when(i + 1 < my_n)
            def _():
                in_copy(i + 1, 1 - slot).start()

            in_copy(i, slot).wait()

            @pl.when(i >= 2)
            def _():
                emb_copy(i - 2, slot).wait()
                for l in range(L - 1):
                    sc_copy(i - 2, slot, l).wait()

            def node_body(n, c2):
                x = [[inb[slot, l, n, pl.ds(g * LANES, LANES)]
                      for g in range(G)] for l in range(L)]

                def red(vs):
                    return _bcast(jnp.sum(_tree(vs)))

                ss = [red([x[l][g] * x[l][g] for g in range(G)])
                      for l in range(L)]
                dq = red([x[0][g] * w2[g] for g in range(G)])
                dm = [red([x[l][g] * w1[g] for g in range(G)])
                      for l in range(1, L)]
                inv = [_rsqrt(jnp.maximum(ss[l], 1e-24)) for l in range(L)]
                sq = dq * inv[0]
                s = [dm[l] * inv[l + 1] + sq + bb for l in range(L - 1)]
                s = [jnp.where(t >= 0, t, 0.01 * t) for t in s]
                mx = jnp.maximum(jnp.maximum(s[0], s[1]), s[2])
                e = [jnp.exp(t - mx) for t in s]
                den = e[0] + e[1] + e[2] + 1e-16
                a = [t / den for t in e]
                c = [a[l] * inv[l + 1] for l in range(L - 1)]
                for g in range(G):
                    embb[slot, n, pl.ds(g * LANES, LANES)] = (
                        x[0][g] * inv[0]
                        + c[0] * x[1][g] + c[1] * x[2][g] + c[2] * x[3][g])
                for l in range(L - 1):
                    plsc.store_compressed(
                        scb.at[pl.ds(slot * SCPAD + l * CHUNK + n, LANES)],
                        a[l], mask=lane_first)
                return c2

            lax.fori_loop(0, CHUNK, node_body, 0)
            emb_copy(i, slot).start()
            for l in range(L - 1):
                sc_copy(i, slot, l).start()
            return carry

        lax.fori_loop(0, my_n, chunk_body, 0)

        @pl.when(my_n >= 2)
        def _():
            s = lax.rem(my_n, 2)
            emb_copy(my_n - 2, s).wait()
            for l in range(L - 1):
                sc_copy(my_n - 2, s, l).wait()

        s = lax.rem(my_n - 1, 2)
        emb_copy(my_n - 1, s).wait()
        for l in range(L - 1):
            sc_copy(my_n - 1, s, l).wait()

    return merge_kernel


def kernel(xs, W_att, b_att):
    L, N, D = xs.shape
    wb = jnp.concatenate(
        [W_att[:, 0], b_att, jnp.zeros((15,), jnp.float32)])
    emb, sc = _make_kernel(L, N, D)(xs, wb)
    return emb, sc


# speedup vs baseline: 1.3128x; 1.0063x over previous
import functools

import jax
import jax.numpy as jnp
from jax import lax
from jax.experimental import pallas as pl
from jax.experimental.pallas import tpu as pltpu
from jax.experimental.pallas import tpu_sc as plsc

LANES = 16
CHUNK = 80
NWORKERS = 32
SCPAD = 3 * CHUNK + LANES


def _rsqrt(v):
    i = lax.bitcast_convert_type(v, jnp.int32)
    i = jnp.int32(0x5F3759DF) - lax.shift_right_logical(i, 1)
    y = lax.bitcast_convert_type(i, jnp.float32)
    for _ in range(2):
        y = y * (1.5 - 0.5 * v * y * y)
    return y


def _tree(vs):
    while len(vs) > 1:
        nxt = [vs[i] + vs[i + 1] for i in range(0, len(vs) - 1, 2)]
        if len(vs) % 2:
            nxt.append(vs[-1])
        vs = nxt
    return vs[0]


def _make_kernel(L, N, D):
    assert D == 128 and L == 4
    G = D // LANES
    nchunks = N // CHUNK
    assert N % CHUNK == 0

    mesh = plsc.VectorSubcoreMesh(core_axis_name="c", subcore_axis_name="s")

    @functools.partial(
        pl.kernel,
        mesh=mesh,
        compiler_params=pltpu.CompilerParams(needs_layout_passes=False),
        out_type=[
            jax.ShapeDtypeStruct((N, D), jnp.float32),
            jax.ShapeDtypeStruct(((L - 1) * N,), jnp.float32),
        ],
        scratch_types=[
            pltpu.VMEM((2, L, CHUNK, D), jnp.float32),
            pltpu.VMEM((2, CHUNK, D), jnp.float32),
            pltpu.VMEM((2 * SCPAD,), jnp.float32),
            pltpu.VMEM((272,), jnp.float32),
            pltpu.SemaphoreType.DMA((2,)),
            pltpu.SemaphoreType.DMA((2,)),
        ],
    )
    def merge_kernel(xs_hbm, wb_hbm, emb_hbm, sc_hbm,
                     inb, embb, scb, wv, sem_in, sem_out):
        wid = lax.axis_index("s") * 2 + lax.axis_index("c")
        pltpu.sync_copy(wb_hbm, wv)
        w1 = [wv[pl.ds(g * LANES, LANES)] for g in range(G)]
        w2 = [wv[pl.ds(D + g * LANES, LANES)] for g in range(G)]
        lane = lax.iota(jnp.int32, LANES)
        lane_first = lane == 0
        half0 = lane < 8
        quart0 = lane < 4
        quart2 = lane < 12
        idx4 = [lane * 0 + (4 * l) for l in range(4)]
        bb = lax.broadcast(wv[pl.ds(2 * D, LANES)][0], (LANES,))
        my_n = (nchunks + (NWORKERS - 1) - wid) // NWORKERS

        def _bcast(s):
            return lax.broadcast(s, (LANES,))

        def chunk_base(i):
            return (wid + i * NWORKERS) * CHUNK

        def in_copy(i, slot):
            return pltpu.make_async_copy(
                xs_hbm.at[:, pl.ds(chunk_base(i), CHUNK)],
                inb.at[slot], sem_in.at[slot])

        def emb_copy(i, slot):
            return pltpu.make_async_copy(
                embb.at[slot], emb_hbm.at[pl.ds(chunk_base(i), CHUNK)],
                sem_out.at[slot])

        def sc_copy(i, slot, l):
            return pltpu.make_async_copy(
                scb.at[pl.ds(slot * SCPAD + l * CHUNK, CHUNK)],
                sc_hbm.at[pl.ds(l * N + chunk_base(i), CHUNK)],
                sem_out.at[slot])

        in_copy(0, 0).start()

        def chunk_body(i, carry):
            slot = lax.rem(i, 2)

            @pl.when(i + 1 < my_n)
            def _():
                in_copy(i + 1, 1 - slot).start()

            in_copy(i, slot).wait()

            @pl.when(i >= 2)
            def _():
                emb_copy(i - 2, slot).wait()
                for l in range(L - 1):
                    sc_copy(i - 2, slot, l).wait()

            def node_body(n, c2):
                x = [[inb[slot, l, n, pl.ds(g * LANES, LANES)]
                      for g in range(G)] for l in range(L)]

                def red(vs):
                    return _bcast(jnp.sum(_tree(vs)))

                ss = [red([x[l][g] * x[l][g] for g in range(G)])
                      for l in range(L)]
                dq = red([x[0][g] * w2[g] for g in range(G)])
                dm = [red([x[l][g] * w1[g] for g in range(G)])
                      for l in range(1, L)]
                u = jnp.where(half0,
                              jnp.where(quart0, ss[0], ss[1]),
                              jnp.where(quart2, ss[2], ss[3]))
                invp = _rsqrt(jnp.maximum(u, 1e-24))
                inv = [jnp.take_along_axis(invp, idx4[l], axis=0)
                       for l in range(L)]
                sq = dq * inv[0]
                s = [dm[l] * inv[l + 1] + sq + bb for l in range(L - 1)]
                s = [jnp.where(t >= 0, t, 0.01 * t) for t in s]
                mx = jnp.maximum(jnp.maximum(s[0], s[1]), s[2])
                e = [jnp.exp(t - mx) for t in s]
                den = e[0] + e[1] + e[2] + 1e-16
                a = [t / den for t in e]
                c = [a[l] * inv[l + 1] for l in range(L - 1)]
                for g in range(G):
                    embb[slot, n, pl.ds(g * LANES, LANES)] = (
                        x[0][g] * inv[0]
                        + c[0] * x[1][g] + c[1] * x[2][g] + c[2] * x[3][g])
                for l in range(L - 1):
                    plsc.store_compressed(
                        scb.at[pl.ds(slot * SCPAD + l * CHUNK + n, LANES)],
                        a[l], mask=lane_first)
                return c2

            lax.fori_loop(0, CHUNK, node_body, 0)
            emb_copy(i, slot).start()
            for l in range(L - 1):
                sc_copy(i, slot, l).start()
            return carry

        lax.fori_loop(0, my_n, chunk_body, 0)

        @pl.when(my_n >= 2)
        def _():
            s = lax.rem(my_n, 2)
            emb_copy(my_n - 2, s).wait()
            for l in range(L - 1):
                sc_copy(my_n - 2, s, l).wait()

        s = lax.rem(my_n - 1, 2)
        emb_copy(my_n - 1, s).wait()
        for l in range(L - 1):
            sc_copy(my_n - 1, s, l).wait()

    return merge_kernel


def kernel(xs, W_att, b_att):
    L, N, D = xs.shape
    wb = jnp.concatenate(
        [W_att[:, 0], b_att, jnp.zeros((15,), jnp.float32)])
    emb, sc = _make_kernel(L, N, D)(xs, wb)
    return emb, sc
